# Initial kernel scaffold; baseline (speedup 1.0000x reference)
import jax
import jax.numpy as jnp
from jax.experimental import pallas as pl


def kernel(x, table, ln_gamma, ln_beta, W, b):
    raise NotImplementedError("write your pallas kernel here")


# speedup vs baseline: 2.9635x; 2.9635x over previous
import functools

import jax
import jax.numpy as jnp
from jax import lax
from jax.experimental import pallas as pl
from jax.experimental.pallas import tpu as pltpu
from jax.experimental.pallas import tpu_sc as plsc

VOCAB = 1000000
D = 32
OUT = 16
B = 16384
SEQ = 50
EPS = 1e-5 * SEQ * SEQ

NC = 2
NS = 16
NW = NC * NS
BPW = B // NW
CHUNK = 128
NCH = BPW // CHUNK


def _sc_pool_body(table_hbm, idx_hbm, out_hbm, idx_v, acc_v, sem):
    wid = lax.axis_index("s") * NC + lax.axis_index("c")

    pltpu.sync_copy(idx_hbm.at[wid], idx_v)

    for c in range(NCH):
        def _zero_row(i, _):
            acc_v[i, pl.ds(0, 16)] = jnp.zeros((16,), jnp.float32)
            acc_v[i, pl.ds(16, 16)] = jnp.zeros((16,), jnp.float32)
            return 0
        lax.fori_loop(0, CHUNK, _zero_row, 0)

        def _fire(j, _):
            pltpu.async_copy(table_hbm.at[idx_v.at[j, c]], acc_v, sem,
                             add=True)
            return 0
        lax.fori_loop(0, SEQ, _fire, 0)

        def _drain(j, _):
            pltpu.make_async_copy(table_hbm.at[idx_v.at[0, c]], acc_v,
                                  sem).wait()
            return 0
        lax.fori_loop(0, SEQ, _drain, 0)

        pltpu.sync_copy(acc_v, out_hbm.at[pl.ds(wid * BPW + c * CHUNK,
                                                CHUNK)])


@functools.partial(jax.jit, static_argnames=())
def _sc_pool(table, idx):
    mesh = plsc.VectorSubcoreMesh(core_axis_name="c", subcore_axis_name="s")
    return pl.kernel(
        _sc_pool_body,
        out_type=jax.ShapeDtypeStruct((B, D), jnp.float32),
        mesh=mesh,
        scratch_types=[
            pltpu.VMEM((SEQ, NCH, CHUNK), jnp.int32),
            pltpu.VMEM((CHUNK, D), jnp.float32),
            pltpu.SemaphoreType.DMA,
        ],
        compiler_params=pltpu.CompilerParams(use_tc_tiling_on_sc=False),
    )(table, idx)


def _head_body(s_ref, g_ref, be_ref, wt_ref, b_ref, o_ref):
    s = s_ref[...]
    mu = jnp.mean(s, axis=1, keepdims=True)
    var = jnp.mean((s - mu) ** 2, axis=1, keepdims=True)
    h = (s - mu) * lax.rsqrt(var + EPS) * g_ref[...] + be_ref[...]
    h = jnp.maximum(h, 0.0)
    o_ref[...] = jnp.dot(h, wt_ref[...],
                         preferred_element_type=jnp.float32) + b_ref[...]


def _head(sums, gamma, beta, Wt, bias):
    blk = 4096
    return pl.pallas_call(
        _head_body,
        grid=(B // blk,),
        in_specs=[
            pl.BlockSpec((blk, D), lambda i: (i, 0)),
            pl.BlockSpec((1, D), lambda i: (0, 0)),
            pl.BlockSpec((1, D), lambda i: (0, 0)),
            pl.BlockSpec((D, OUT), lambda i: (0, 0)),
            pl.BlockSpec((1, OUT), lambda i: (0, 0)),
        ],
        out_specs=pl.BlockSpec((blk, OUT), lambda i: (i, 0)),
        out_shape=jax.ShapeDtypeStruct((B, OUT), jnp.float32),
    )(sums, gamma, beta, Wt, bias)


def kernel(x, table, ln_gamma, ln_beta, W, b):
    idx = x.reshape(NW, NCH, CHUNK, SEQ).transpose(0, 3, 1, 2)
    sums = _sc_pool(table, idx)
    return _head(sums, ln_gamma.reshape(1, D), ln_beta.reshape(1, D),
                 W.T, b.reshape(1, OUT))


# speedup vs baseline: 2.9772x; 1.0046x over previous
import functools

import jax
import jax.numpy as jnp
from jax import lax
from jax.experimental import pallas as pl
from jax.experimental.pallas import tpu as pltpu
from jax.experimental.pallas import tpu_sc as plsc

VOCAB = 1000000
D = 32
OUT = 16
B = 16384
SEQ = 50
EPS = 1e-5 * SEQ * SEQ

NC = 2
NS = 16
NW = NC * NS
BPW = B // NW
CHUNK = 128
NCH = BPW // CHUNK


def _sc_pool_body(table_hbm, idx_hbm, out_hbm, idx_v, acc_v, isem, sem):
    wid = lax.axis_index("s") * NC + lax.axis_index("c")

    idx_cp = pltpu.async_copy(idx_hbm.at[wid], idx_v, isem)

    def _zero_row(i, _):
        acc_v[i, pl.ds(0, 16)] = jnp.zeros((16,), jnp.float32)
        acc_v[i, pl.ds(16, 16)] = jnp.zeros((16,), jnp.float32)
        return 0
    lax.fori_loop(0, BPW, _zero_row, 0)
    idx_cp.wait()

    for c in range(NCH):
        def _fire(j, _, c=c):
            pltpu.async_copy(table_hbm.at[idx_v.at[j, c]],
                             acc_v.at[pl.ds(c * CHUNK, CHUNK)], sem,
                             add=True)
            return 0
        lax.fori_loop(0, SEQ, _fire, 0)

    def _drain(t, _):
        pltpu.make_async_copy(table_hbm.at[idx_v.at[0, 0]],
                              acc_v.at[pl.ds(0, CHUNK)], sem).wait()
        return 0
    lax.fori_loop(0, SEQ * NCH, _drain, 0)

    pltpu.sync_copy(acc_v, out_hbm.at[pl.ds(wid * BPW, BPW)])


@functools.partial(jax.jit, static_argnames=())
def _sc_pool(table, idx):
    mesh = plsc.VectorSubcoreMesh(core_axis_name="c", subcore_axis_name="s")
    return pl.kernel(
        _sc_pool_body,
        out_type=jax.ShapeDtypeStruct((B, D), jnp.float32),
        mesh=mesh,
        scratch_types=[
            pltpu.VMEM((SEQ, NCH, CHUNK), jnp.int32),
            pltpu.VMEM((BPW, D), jnp.float32),
            pltpu.SemaphoreType.DMA,
            pltpu.SemaphoreType.DMA,
        ],
        compiler_params=pltpu.CompilerParams(use_tc_tiling_on_sc=False),
    )(table, idx)


def _head_body(s_ref, g_ref, be_ref, wt_ref, b_ref, o_ref):
    s = s_ref[...]
    mu = jnp.mean(s, axis=1, keepdims=True)
    var = jnp.mean((s - mu) ** 2, axis=1, keepdims=True)
    h = (s - mu) * lax.rsqrt(var + EPS) * g_ref[...] + be_ref[...]
    h = jnp.maximum(h, 0.0)
    o_ref[...] = jnp.dot(h, wt_ref[...],
                         preferred_element_type=jnp.float32) + b_ref[...]


def _head(sums, gamma, beta, Wt, bias):
    blk = 4096
    return pl.pallas_call(
        _head_body,
        grid=(B // blk,),
        in_specs=[
            pl.BlockSpec((blk, D), lambda i: (i, 0)),
            pl.BlockSpec((1, D), lambda i: (0, 0)),
            pl.BlockSpec((1, D), lambda i: (0, 0)),
            pl.BlockSpec((D, OUT), lambda i: (0, 0)),
            pl.BlockSpec((1, OUT), lambda i: (0, 0)),
        ],
        out_specs=pl.BlockSpec((blk, OUT), lambda i: (i, 0)),
        out_shape=jax.ShapeDtypeStruct((B, OUT), jnp.float32),
    )(sums, gamma, beta, Wt, bias)


def kernel(x, table, ln_gamma, ln_beta, W, b):
    idx = x.reshape(NW, NCH, CHUNK, SEQ).transpose(0, 3, 1, 2)
    sums = _sc_pool(table, idx)
    return _head(sums, ln_gamma.reshape(1, D), ln_beta.reshape(1, D),
                 W.T, b.reshape(1, OUT))
